# baseline (speedup 1.0000x reference)
import jax
import jax.numpy as jnp
from jax import lax
from jax.experimental import pallas as pl
from jax.experimental.pallas import tpu as pltpu
from jax.experimental.pallas import tpu_sc as plsc

_N = 10000
_E = 160000
_NCORES = 2
_NTILES = 16
_NACC = 10016
_IPT = _NACC // _NTILES
_OPT = _N // _NTILES


def _make_segsum(width, nch, ch, with_counts):
  out_types = [jax.ShapeDtypeStruct((_NCORES, _N, width), jnp.float32)]
  scratch = [
      pltpu.VMEM_SHARED((_NACC, width), jnp.float32),
      pltpu.VMEM((nch, ch), jnp.int32),
      pltpu.VMEM((nch, ch), jnp.int32),
      pltpu.VMEM((ch, width), jnp.float32),
      pltpu.VMEM((ch, width), jnp.float32),
      pltpu.SemaphoreType.DMA,
      pltpu.SemaphoreType.DMA,
  ]
  if with_counts:
    out_types.append(jax.ShapeDtypeStruct((_N, 16), jnp.float32))
    scratch += [
        pltpu.VMEM_SHARED((_NACC, 16), jnp.float32),
        pltpu.VMEM((ch, 16), jnp.float32),
    ]
  mesh = plsc.VectorSubcoreMesh(core_axis_name="c", subcore_axis_name="s")

  def body(*refs):
    it = iter(refs)
    table = next(it)
    src_hbm = next(it)
    dst_hbm = next(it)
    zeros_hbm = next(it)
    if with_counts:
      zcnt_hbm = next(it)
      ones_hbm = next(it)
    msg_hbm = next(it)
    if with_counts:
      cnt_hbm = next(it)
    acc_sh = next(it)
    src_v = next(it)
    dst_v = next(it)
    rows = (next(it), next(it))
    sems = (next(it), next(it))
    if with_counts:
      cnt_sh = next(it)
      ones_v = next(it)

    c = lax.axis_index("c")
    s = lax.axis_index("s")

    pltpu.sync_copy(zeros_hbm, acc_sh.at[pl.ds(s * _IPT, _IPT)])
    if with_counts:
      @pl.when(c == 0)
      def _():
        pltpu.sync_copy(zcnt_hbm, cnt_sh.at[pl.ds(s * _IPT, _IPT)])
      pltpu.sync_copy(ones_hbm, ones_v)
    pltpu.sync_copy(src_hbm.at[c, s], src_v)
    pltpu.sync_copy(dst_hbm.at[c, s], dst_v)
    plsc.subcore_barrier()

    def start_gather(j, b):
      pltpu.async_copy(table.at[src_v.at[j]], rows[b], sems[b])

    def consume(j, b):
      pltpu.make_async_copy(table.at[src_v.at[j]], rows[b], sems[b]).wait()
      nxt = j + 2

      @pl.when(nxt < nch)
      def _():
        start_gather(nxt, b)

      pltpu.sync_copy(rows[b], acc_sh.at[dst_v.at[j]], add=True)
      if with_counts:
        @pl.when(c == 0)
        def _():
          pltpu.sync_copy(ones_v, cnt_sh.at[dst_v.at[j]], add=True)

    start_gather(0, 0)
    start_gather(1, 1)

    @pl.loop(0, nch, step=2)
    def _(k):
      for b in range(2):
        consume(k + b, b)

    plsc.subcore_barrier()
    pltpu.sync_copy(acc_sh.at[pl.ds(s * _OPT, _OPT)],
                    msg_hbm.at[c, pl.ds(s * _OPT, _OPT)])
    if with_counts:
      @pl.when(c == 0)
      def _():
        pltpu.sync_copy(cnt_sh.at[pl.ds(s * _OPT, _OPT)],
                        cnt_hbm.at[pl.ds(s * _OPT, _OPT)])

  return pl.kernel(body, out_type=tuple(out_types), mesh=mesh,
                   scratch_types=scratch,
                   compiler_params=pltpu.CompilerParams(
                       use_tc_tiling_on_sc=False))


_EPAD0 = _NTILES * 160 * 64
_EPAD1 = _NTILES * 40 * 128
_segsum0 = _make_segsum(width=128, nch=160, ch=64, with_counts=True)
_segsum1 = _make_segsum(width=48, nch=40, ch=128, with_counts=False)

_RB = 1000


def _dense_xw_body(x_ref, wr_ref, b0_ref, xw_ref):
  xw_ref[...] = jnp.dot(x_ref[...], wr_ref[...],
                        preferred_element_type=jnp.float32) + b0_ref[...]


_dense_xw = pl.pallas_call(
    _dense_xw_body,
    grid=(_N // _RB,),
    in_specs=[
        pl.BlockSpec((_RB, 256), lambda i: (i, 0)),
        pl.BlockSpec((256, 256), lambda i: (0, 0)),
        pl.BlockSpec((1, 256), lambda i: (0, 0)),
    ],
    out_specs=pl.BlockSpec((_RB, 256), lambda i: (i, 0)),
    out_shape=jax.ShapeDtypeStruct((_N, 256), jnp.float32),
)


def _dense0_body(msg_ref, cnt_ref, xw_ref, wa_ref, w2a_ref, w2b_ref,
                 y1_ref, y2_ref):
  acc = jnp.dot(msg_ref[0], wa_ref[:128, :], preferred_element_type=jnp.float32)
  acc = acc + jnp.dot(msg_ref[1], wa_ref[128:, :],
                      preferred_element_type=jnp.float32)
  cnt = jnp.max(cnt_ref[...], axis=1, keepdims=True)
  inv = 1.0 / jnp.maximum(cnt, 1.0)
  x1 = jnp.maximum(acc * inv + xw_ref[...], 0.0)
  y1_ref[...] = jnp.dot(x1, w2a_ref[...], preferred_element_type=jnp.float32)
  y2_ref[...] = jnp.dot(x1, w2b_ref[...], preferred_element_type=jnp.float32)


_dense0 = pl.pallas_call(
    _dense0_body,
    grid=(_N // _RB,),
    in_specs=[
        pl.BlockSpec((_NCORES, _RB, 128), lambda i: (0, i, 0)),
        pl.BlockSpec((_RB, 16), lambda i: (i, 0)),
        pl.BlockSpec((_RB, 256), lambda i: (i, 0)),
        pl.BlockSpec((256, 256), lambda i: (0, 0)),
        pl.BlockSpec((256, 48), lambda i: (0, 0)),
        pl.BlockSpec((256, 48), lambda i: (0, 0)),
    ],
    out_specs=[
        pl.BlockSpec((_RB, 48), lambda i: (i, 0)),
        pl.BlockSpec((_RB, 48), lambda i: (i, 0)),
    ],
    out_shape=[
        jax.ShapeDtypeStruct((_N, 48), jnp.float32),
        jax.ShapeDtypeStruct((_N, 48), jnp.float32),
    ],
)


def _dense1_body(msg_ref, cnt_ref, y2_ref, c1_ref, out_ref):
  ssum = msg_ref[0] + msg_ref[1]
  cnt = jnp.max(cnt_ref[...], axis=1, keepdims=True)
  inv = 1.0 / jnp.maximum(cnt, 1.0)
  res = ssum * inv + y2_ref[...] + c1_ref[...]
  out_ref[...] = res[:, :40]


_dense1 = pl.pallas_call(
    _dense1_body,
    grid=(_N // _RB,),
    in_specs=[
        pl.BlockSpec((_NCORES, _RB, 48), lambda i: (0, i, 0)),
        pl.BlockSpec((_RB, 16), lambda i: (i, 0)),
        pl.BlockSpec((_RB, 48), lambda i: (i, 0)),
        pl.BlockSpec((1, 48), lambda i: (0, 0)),
    ],
    out_specs=pl.BlockSpec((_RB, 40), lambda i: (i, 0)),
    out_shape=jax.ShapeDtypeStruct((_N, 40), jnp.float32),
)


def kernel(x, edge_index, sage0_Wl, sage0_bl, sage0_Wr, lin0_W, lin0_b,
           sage1_Wl, sage1_bl, sage1_Wr, lin1_W, lin1_b):
  src = edge_index[0].astype(jnp.int32)
  dst = edge_index[1].astype(jnp.int32)

  x2 = x.reshape(2 * _N, 128)
  p0 = _EPAD0 - _E
  src0 = jnp.concatenate([src * 2, jnp.zeros((p0,), jnp.int32)])
  dst0 = jnp.concatenate([dst, jnp.full((p0,), _N, jnp.int32)])
  srcA = jnp.stack([src0, src0 + 1]).reshape(_NCORES, _NTILES, 160, 64)
  dstA = jnp.broadcast_to(
      dst0.reshape(1, _NTILES, 160, 64), (_NCORES, _NTILES, 160, 64))
  zeros128 = jnp.zeros((_IPT, 128), jnp.float32)
  zeros16 = jnp.zeros((_IPT, 16), jnp.float32)
  ones = jnp.ones((64, 16), jnp.float32)
  msg0, cnt = _segsum0(x2, srcA, dstA, zeros128, zeros16, ones)

  b0 = (sage0_bl + lin0_b).reshape(1, 256)
  xw = _dense_xw(x, sage0_Wr.T, b0)
  w2a = jnp.pad(sage1_Wl.T, ((0, 0), (0, 8)))
  w2b = jnp.pad(sage1_Wr.T, ((0, 0), (0, 8)))
  y1, y2 = _dense0(msg0, cnt, xw, sage0_Wl.T, w2a, w2b)

  half = _E // 2
  p1 = _EPAD1 - half
  z1 = jnp.zeros((p1,), jnp.int32)
  d1 = jnp.full((p1,), _N, jnp.int32)
  srcC = jnp.stack([jnp.concatenate([src[:half], z1]),
                    jnp.concatenate([src[half:], z1])]).reshape(
                        _NCORES, _NTILES, 40, 128)
  dstC = jnp.stack([jnp.concatenate([dst[:half], d1]),
                    jnp.concatenate([dst[half:], d1])]).reshape(
                        _NCORES, _NTILES, 40, 128)
  zeros48 = jnp.zeros((_IPT, 48), jnp.float32)
  (msg1,) = _segsum1(y1, srcC, dstC, zeros48)

  c1 = (sage1_bl + lin1_W @ lin0_b + lin1_b)
  c1p = jnp.pad(c1, (0, 8)).reshape(1, 48)
  return _dense1(msg1, cnt, y2, c1p)


# speedup vs baseline: 1.0870x; 1.0870x over previous
import jax
import jax.numpy as jnp
from jax import lax
from jax.experimental import pallas as pl
from jax.experimental.pallas import tpu as pltpu
from jax.experimental.pallas import tpu_sc as plsc

_N = 10000
_E = 160000
_NCORES = 2
_NTILES = 16
_NACC = 10016
_IPT = _NACC // _NTILES
_OPT = _N // _NTILES


def _make_segsum(width, nch, ch, with_counts):
  out_types = [jax.ShapeDtypeStruct((_NCORES, _N, width), jnp.float32)]
  scratch = [
      pltpu.VMEM_SHARED((_NACC, width), jnp.float32),
      pltpu.VMEM((nch, ch), jnp.int32),
      pltpu.VMEM((nch, ch), jnp.int32),
      pltpu.VMEM((ch, width), jnp.float32),
      pltpu.VMEM((ch, width), jnp.float32),
      pltpu.SemaphoreType.DMA,
      pltpu.SemaphoreType.DMA,
  ]
  if with_counts:
    out_types.append(jax.ShapeDtypeStruct((_N, 16), jnp.float32))
    scratch += [
        pltpu.VMEM_SHARED((_NACC, 16), jnp.float32),
        pltpu.VMEM((ch, 16), jnp.float32),
    ]
  mesh = plsc.VectorSubcoreMesh(core_axis_name="c", subcore_axis_name="s")

  def body(*refs):
    it = iter(refs)
    table = next(it)
    src_hbm = next(it)
    dst_hbm = next(it)
    zeros_hbm = next(it)
    if with_counts:
      zcnt_hbm = next(it)
      ones_hbm = next(it)
    msg_hbm = next(it)
    if with_counts:
      cnt_hbm = next(it)
    acc_sh = next(it)
    src_v = next(it)
    dst_v = next(it)
    rows = (next(it), next(it))
    sems = (next(it), next(it))
    if with_counts:
      cnt_sh = next(it)
      ones_v = next(it)

    c = lax.axis_index("c")
    s = lax.axis_index("s")

    pltpu.sync_copy(zeros_hbm, acc_sh.at[pl.ds(s * _IPT, _IPT)])
    if with_counts:
      @pl.when(c == 0)
      def _():
        pltpu.sync_copy(zcnt_hbm, cnt_sh.at[pl.ds(s * _IPT, _IPT)])
      pltpu.sync_copy(ones_hbm, ones_v)
    pltpu.sync_copy(src_hbm.at[c, s], src_v)
    pltpu.sync_copy(dst_hbm.at[c, s], dst_v)
    plsc.subcore_barrier()

    def start_gather(j, b):
      pltpu.async_copy(table.at[src_v.at[j]], rows[b], sems[b])

    def consume(j, b):
      pltpu.make_async_copy(table.at[src_v.at[j]], rows[b], sems[b]).wait()
      nxt = j + 2

      @pl.when(nxt < nch)
      def _():
        start_gather(nxt, b)

      pltpu.sync_copy(rows[b], acc_sh.at[dst_v.at[j]], add=True)
      if with_counts:
        @pl.when(c == 0)
        def _():
          pltpu.sync_copy(ones_v, cnt_sh.at[dst_v.at[j]], add=True)

    start_gather(0, 0)
    start_gather(1, 1)

    @pl.loop(0, nch, step=2)
    def _(k):
      for b in range(2):
        consume(k + b, b)

    plsc.subcore_barrier()
    pltpu.sync_copy(acc_sh.at[pl.ds(s * _OPT, _OPT)],
                    msg_hbm.at[c, pl.ds(s * _OPT, _OPT)])
    if with_counts:
      @pl.when(c == 0)
      def _():
        pltpu.sync_copy(cnt_sh.at[pl.ds(s * _OPT, _OPT)],
                        cnt_hbm.at[pl.ds(s * _OPT, _OPT)])

  return pl.kernel(body, out_type=tuple(out_types), mesh=mesh,
                   scratch_types=scratch,
                   compiler_params=pltpu.CompilerParams(
                       use_tc_tiling_on_sc=False))


_EPAD0 = _NTILES * 160 * 64
_EPAD1 = _NTILES * 40 * 128
_segsum0 = _make_segsum(width=128, nch=160, ch=64, with_counts=True)
_segsum1 = _make_segsum(width=48, nch=40, ch=128, with_counts=False)

_RB = 1000


def _dense_xw_body(x_ref, wr_ref, b0_ref, xw_ref):
  xw_ref[...] = jnp.dot(x_ref[...], wr_ref[...],
                        preferred_element_type=jnp.float32) + b0_ref[...]


_dense_xw = pl.pallas_call(
    _dense_xw_body,
    grid=(_N // _RB,),
    in_specs=[
        pl.BlockSpec((_RB, 256), lambda i: (i, 0)),
        pl.BlockSpec((256, 256), lambda i: (0, 0)),
        pl.BlockSpec((1, 256), lambda i: (0, 0)),
    ],
    out_specs=pl.BlockSpec((_RB, 256), lambda i: (i, 0)),
    out_shape=jax.ShapeDtypeStruct((_N, 256), jnp.float32),
)


def _dense0_body(msg_ref, cnt_ref, xw_ref, wa_ref, w2a_ref, w2b_ref,
                 y1_ref, y2_ref):
  acc = jnp.dot(msg_ref[0], wa_ref[:128, :], preferred_element_type=jnp.float32)
  acc = acc + jnp.dot(msg_ref[1], wa_ref[128:, :],
                      preferred_element_type=jnp.float32)
  cnt = jnp.max(cnt_ref[...], axis=1, keepdims=True)
  inv = 1.0 / jnp.maximum(cnt, 1.0)
  x1 = jnp.maximum(acc * inv + xw_ref[...], 0.0)
  y1_ref[...] = jnp.dot(x1, w2a_ref[...], preferred_element_type=jnp.float32)
  y2_ref[...] = jnp.dot(x1, w2b_ref[...], preferred_element_type=jnp.float32)


_dense0 = pl.pallas_call(
    _dense0_body,
    grid=(_N // _RB,),
    in_specs=[
        pl.BlockSpec((_NCORES, _RB, 128), lambda i: (0, i, 0)),
        pl.BlockSpec((_RB, 16), lambda i: (i, 0)),
        pl.BlockSpec((_RB, 256), lambda i: (i, 0)),
        pl.BlockSpec((256, 256), lambda i: (0, 0)),
        pl.BlockSpec((256, 48), lambda i: (0, 0)),
        pl.BlockSpec((256, 48), lambda i: (0, 0)),
    ],
    out_specs=[
        pl.BlockSpec((_RB, 48), lambda i: (i, 0)),
        pl.BlockSpec((_RB, 48), lambda i: (i, 0)),
    ],
    out_shape=[
        jax.ShapeDtypeStruct((_N, 48), jnp.float32),
        jax.ShapeDtypeStruct((_N, 48), jnp.float32),
    ],
)


def _dense1_body(msg_ref, cnt_ref, y2_ref, c1_ref, out_ref):
  ssum = msg_ref[0] + msg_ref[1]
  cnt = jnp.max(cnt_ref[...], axis=1, keepdims=True)
  inv = 1.0 / jnp.maximum(cnt, 1.0)
  res = ssum * inv + y2_ref[...] + c1_ref[...]
  out_ref[...] = res[:, :40]


_dense1 = pl.pallas_call(
    _dense1_body,
    grid=(_N // _RB,),
    in_specs=[
        pl.BlockSpec((_NCORES, _RB, 48), lambda i: (0, i, 0)),
        pl.BlockSpec((_RB, 16), lambda i: (i, 0)),
        pl.BlockSpec((_RB, 48), lambda i: (i, 0)),
        pl.BlockSpec((1, 48), lambda i: (0, 0)),
    ],
    out_specs=pl.BlockSpec((_RB, 40), lambda i: (i, 0)),
    out_shape=jax.ShapeDtypeStruct((_N, 40), jnp.float32),
)


def kernel(x, edge_index, sage0_Wl, sage0_bl, sage0_Wr, lin0_W, lin0_b,
           sage1_Wl, sage1_bl, sage1_Wr, lin1_W, lin1_b):
  src = edge_index[0].astype(jnp.int32)
  dst = edge_index[1].astype(jnp.int32)

  x2 = x.reshape(2 * _N, 128)
  p0 = (_EPAD0 - _E) // _NTILES
  dum0 = jnp.broadcast_to(_N + (jnp.arange(p0, dtype=jnp.int32) % 16),
                          (_NTILES, p0))
  src0 = jnp.concatenate(
      [src.reshape(_NTILES, -1) * 2,
       jnp.zeros((_NTILES, p0), jnp.int32)], axis=1)
  dst0 = jnp.concatenate([dst.reshape(_NTILES, -1), dum0], axis=1)
  srcA = jnp.stack([src0, src0 + 1]).reshape(_NCORES, _NTILES, 160, 64)
  dstA = jnp.broadcast_to(
      dst0.reshape(1, _NTILES, 160, 64), (_NCORES, _NTILES, 160, 64))
  zeros128 = jnp.zeros((_IPT, 128), jnp.float32)
  zeros16 = jnp.zeros((_IPT, 16), jnp.float32)
  ones = jnp.ones((64, 16), jnp.float32)
  msg0, cnt = _segsum0(x2, srcA, dstA, zeros128, zeros16, ones)

  b0 = (sage0_bl + lin0_b).reshape(1, 256)
  xw = _dense_xw(x, sage0_Wr.T, b0)
  w2a = jnp.pad(sage1_Wl.T, ((0, 0), (0, 8)))
  w2b = jnp.pad(sage1_Wr.T, ((0, 0), (0, 8)))
  y1, y2 = _dense0(msg0, cnt, xw, sage0_Wl.T, w2a, w2b)

  p1 = (_EPAD1 - _E // 2) // _NTILES
  dum1 = jnp.broadcast_to(_N + (jnp.arange(p1, dtype=jnp.int32) % 16),
                          (_NCORES, _NTILES, p1))
  srcC = jnp.concatenate(
      [src.reshape(_NCORES, _NTILES, -1),
       jnp.zeros((_NCORES, _NTILES, p1), jnp.int32)], axis=2).reshape(
          _NCORES, _NTILES, 40, 128)
  dstC = jnp.concatenate(
      [dst.reshape(_NCORES, _NTILES, -1), dum1], axis=2).reshape(
          _NCORES, _NTILES, 40, 128)
  zeros48 = jnp.zeros((_IPT, 48), jnp.float32)
  (msg1,) = _segsum1(y1, srcC, dstC, zeros48)

  c1 = (sage1_bl + lin1_W @ lin0_b + lin1_b)
  c1p = jnp.pad(c1, (0, 8)).reshape(1, 48)
  return _dense1(msg1, cnt, y2, c1p)
